# baseline (speedup 1.0000x reference)
import functools

import jax
import jax.numpy as jnp
from jax import lax
from jax.experimental import pallas as pl
from jax.experimental.pallas import tpu as pltpu
from jax.experimental.pallas import tpu_sc as plsc

_NUM_CORES = 2
_NUM_SUBCORES = 16
_LANES = 16
_CHUNK = 128


def _embedding_bag_sc(indices_t, table_lin, batch, hist, dim):
    n_workers = _NUM_CORES * _NUM_SUBCORES
    bags_per_w = batch // n_workers
    bags_per_core = bags_per_w * _NUM_SUBCORES

    mesh = plsc.VectorSubcoreMesh(core_axis_name="c", subcore_axis_name="s")

    @functools.partial(
        pl.kernel,
        out_type=jax.ShapeDtypeStruct((batch, dim), jnp.float32),
        mesh=mesh,
        scratch_types=[
            pltpu.VMEM((hist, _CHUNK), jnp.int32),
            pltpu.VMEM((_CHUNK,), jnp.int32),
            pltpu.VMEM((8, _CHUNK, dim), jnp.float32),
            pltpu.VMEM_SHARED((bags_per_core, dim), jnp.float32),
            [pltpu.SemaphoreType.DMA] * 8,
            [pltpu.SemaphoreType.DMA] * 8,
        ],
        compiler_params=pltpu.CompilerParams(use_tc_tiling_on_sc=False),
    )
    def ker(idx_hbm, table_hbm, out_hbm, idx_v, bag_v, rows_v, acc_sh,
            gsems, ssems):
        c = lax.axis_index("c")
        s = lax.axis_index("s")
        w = c * _NUM_SUBCORES + s
        b0 = w * bags_per_w

        pltpu.sync_copy(idx_hbm.at[:, pl.ds(b0, bags_per_w)], idx_v)

        sbase = s * bags_per_w
        for j in range(0, _CHUNK, _LANES):
            bag_v[pl.ds(j, _LANES)] = lax.iota(jnp.int32, _LANES) + (
                sbase + j)

        @pl.loop(0, _CHUNK)
        def _(b):
            for d in range(0, dim, _LANES):
                rows_v[0, b, pl.ds(d, _LANES)] = jnp.zeros((_LANES,),
                                                           jnp.float32)
        for z in range(0, bags_per_w, _CHUNK):
            pltpu.sync_copy(rows_v.at[0],
                            acc_sh.at[pl.ds(sbase + z, _CHUNK)])

        depth, ahead = 8, 3
        gd = [None] * depth
        sd = [None] * depth
        for j in range(ahead):
            gd[j] = pltpu.async_copy(
                table_hbm.at[idx_v.at[j]], rows_v.at[j], gsems[j])
        for l in range(hist):
            b = l % depth
            gd[b].wait()
            sd[b] = pltpu.async_copy(
                rows_v.at[b], acc_sh.at[bag_v], ssems[b], add=True)
            nxt = l + ahead
            if nxt < hist:
                nb = nxt % depth
                if nxt >= depth:
                    sd[nb].wait()
                gd[nb] = pltpu.async_copy(
                    table_hbm.at[idx_v.at[nxt]], rows_v.at[nb], gsems[nb])
        for b in range(depth):
            sd[b].wait()

        pltpu.sync_copy(acc_sh.at[pl.ds(sbase, bags_per_w)],
                        out_hbm.at[pl.ds(b0, bags_per_w)])

    return ker(indices_t, table_lin)


def _relayout_tc(table_t, vocab, dim):
    packs = 128 // dim
    cols = 24576
    per_q = (vocab + packs * cols - 1) // (packs * cols)
    stride = per_q * cols

    def body(*refs):
        o_ref = refs[packs]
        stacked = jnp.concatenate([refs[m][...] for m in range(packs)],
                                  axis=0)
        o_ref[...] = stacked.T

    return pl.pallas_call(
        body,
        grid=(per_q,),
        in_specs=[pl.BlockSpec((dim, cols),
                               functools.partial(
                                   lambda q, i: (0, jnp.minimum(
                                       q * per_q + i,
                                       (vocab - 1) // cols)), m))
                  for m in range(packs)],
        out_specs=pl.BlockSpec((cols, 128), lambda i: (i, 0)),
        out_shape=jax.ShapeDtypeStruct((stride, 128), jnp.float32),
        compiler_params=pltpu.CompilerParams(
            dimension_semantics=("arbitrary",)),
    )(*([table_t] * packs)), stride


def kernel(indices, table):
    batch, hist = indices.shape
    vocab, dim = table.shape
    n_workers = _NUM_CORES * _NUM_SUBCORES
    assert batch % (n_workers * _CHUNK) == 0 and dim % _LANES == 0
    assert batch // n_workers == _CHUNK
    assert 128 % dim == 0
    packs = 128 // dim

    table_pack, stride = _relayout_tc(table.astype(jnp.float32).T, vocab,
                                      dim)
    it = indices.T.astype(jnp.int32)
    indices_t = (it % stride) * packs + it // stride
    table_lin = table_pack.reshape(table_pack.shape[0] * packs, dim)
    return _embedding_bag_sc(indices_t, table_lin, batch, hist, dim)
